# baseline (speedup 1.0000x reference)
import functools

import jax
import jax.numpy as jnp
from jax import lax
from jax.experimental import pallas as pl
from jax.experimental.pallas import tpu as pltpu
from jax.experimental.pallas import tpu_sc as plsc

N_NODES = 10000
N_EDGES = 320000
D_FEAT = 128

NC = 2
NS = 16
NW = NC * NS
RANGE = 320
TRASH = RANGE
CHUNK = 6400
NCHUNKS = N_EDGES // CHUNK
NBLOCKS_OUT = RANGE // 16
UNROLL = 5
NRING = 8

_NEG_INF = float("-inf")


def _seg_max_kernel(feats_hbm, src_hbm, dst_hbm, out_hbm,
                    shd0, shs0, shd1, shs1, dbuf, sbuf, sel, ring, acc,
                    sem, sem_e0, sem_e1):
    sid = lax.axis_index("s")
    wid = sid * NC + lax.axis_index("c")
    lo = wid * RANGE
    hi = lo + RANGE
    slc = CHUNK // NS

    neg = jnp.full((16,), _NEG_INF, jnp.float32)

    def init_body(r, _):
        for c in range(D_FEAT // 16):
            acc[r, pl.ds(c * 16, 16)] = neg
        return 0

    lax.fori_loop(0, RANGE + 1, init_body, 0)

    dummy_packed = jnp.full((16,), TRASH, jnp.int32)

    def fire_chunk(ch, shd, shs, sem_e):
        @pl.when(sid == 0)
        def _():
            base = ch * CHUNK
            pltpu.async_copy(dst_hbm.at[pl.ds(base, CHUNK)], shd, sem_e)
            pltpu.async_copy(src_hbm.at[pl.ds(base, CHUNK)], shs, sem_e)

    def wait_chunk(shd, shs, sem_e):
        @pl.when(sid == 0)
        def _():
            pltpu.make_async_copy(dst_hbm.at[pl.ds(0, CHUNK)],
                                  shd, sem_e).wait()
            pltpu.make_async_copy(src_hbm.at[pl.ds(0, CHUNK)],
                                  shs, sem_e).wait()
        plsc.subcore_barrier()
        pltpu.sync_copy(shd, dbuf)
        pltpu.sync_copy(shs, sbuf)
        plsc.subcore_barrier()

    def chunk_body(ch, shd, shs, sem_e):
        wait_chunk(shd, shs, sem_e)

        def scan_body(i, kv):
            for u in range(UNROLL):
                off = (i * UNROLL + u) * 16
                d = dbuf[pl.ds(off, 16)]
                m = (d >= lo) & (d < hi)
                s = sbuf[pl.ds(off, 16)]
                packed = s * 512 + (d - lo)
                cs = plsc.cumsum(jnp.where(m, 1, 0))
                pos = (kv - 1) + cs
                plsc.store_scatter(sel, [pos], packed, mask=m)
                kv = kv + plsc.all_reduce_population_count(m)
            return kv

        kv = lax.fori_loop(0, CHUNK // 16 // UNROLL, scan_body,
                           jnp.zeros((16,), jnp.int32))
        k = kv[0]

        pad_pos = k + lax.iota(jnp.int32, 16)
        plsc.store_scatter(sel, [pad_pos], dummy_packed)
        nblk = (k + 15) // 16

        def fire(j):
            iv = lax.shift_right_logical(sel[pl.ds(j * 16, 16)], 9)
            pltpu.async_copy(feats_hbm.at[iv], ring.at[j % NRING], sem)

        def prime(j, _):
            @pl.when(j < nblk)
            def _():
                fire(j)
            return 0

        lax.fori_loop(0, NRING, prime, 0)

        def drain_body(j, _):
            slot = j % NRING
            pltpu.make_async_copy(feats_hbm.at[pl.ds(0, 16)],
                                  ring.at[slot], sem).wait()
            ldv = sel[pl.ds(j * 16, 16)] & 511
            lds = [ldv[r] for r in range(16)]
            for r in range(16):
                ld = lds[r]
                for c in range(D_FEAT // 16):
                    sl = pl.ds(c * 16, 16)
                    acc[ld, sl] = jnp.maximum(acc[ld, sl], ring[slot, r, sl])

            @pl.when(j + NRING < nblk)
            def _():
                fire(j + NRING)

            return 0

        lax.fori_loop(0, nblk, drain_body, 0)

    fire_chunk(0, shd0, shs0, sem_e0)

    def outer_body(jj, _):
        ch0 = 2 * jj

        @pl.when(ch0 + 1 < NCHUNKS)
        def _():
            fire_chunk(ch0 + 1, shd1, shs1, sem_e1)

        chunk_body(ch0, shd0, shs0, sem_e0)

        @pl.when(ch0 + 2 < NCHUNKS)
        def _():
            fire_chunk(ch0 + 2, shd0, shs0, sem_e0)

        @pl.when(ch0 + 1 < NCHUNKS)
        def _():
            chunk_body(ch0 + 1, shd1, shs1, sem_e1)

        return 0

    lax.fori_loop(0, (NCHUNKS + 1) // 2, outer_body, 0)

    def fix_body(r, _):
        for c in range(D_FEAT // 16):
            sl = pl.ds(c * 16, 16)
            v = acc[r, sl]
            acc[r, sl] = jnp.where(v == _NEG_INF, 0.0, v)
        return 0

    lax.fori_loop(0, RANGE, fix_body, 0)

    def out_body(b, _):
        @pl.when(lo + b * 16 < N_NODES)
        def _():
            pltpu.sync_copy(acc.at[pl.ds(b * 16, 16)],
                            out_hbm.at[pl.ds(lo + b * 16, 16)])
        return 0

    lax.fori_loop(0, NBLOCKS_OUT, out_body, 0)


@jax.jit
def _seg_max(node_feats, src, dst):
    mesh = plsc.VectorSubcoreMesh(core_axis_name="c", subcore_axis_name="s")
    f = functools.partial(
        pl.kernel,
        mesh=mesh,
        out_type=jax.ShapeDtypeStruct((N_NODES, D_FEAT), jnp.float32),
        scratch_types=[
            pltpu.VMEM_SHARED((CHUNK,), jnp.int32),
            pltpu.VMEM_SHARED((CHUNK,), jnp.int32),
            pltpu.VMEM_SHARED((CHUNK,), jnp.int32),
            pltpu.VMEM_SHARED((CHUNK,), jnp.int32),
            pltpu.VMEM((CHUNK,), jnp.int32),
            pltpu.VMEM((CHUNK,), jnp.int32),
            pltpu.VMEM((CHUNK + 16,), jnp.int32),
            pltpu.VMEM((NRING, 16, D_FEAT), jnp.float32),
            pltpu.VMEM((RANGE + 1, D_FEAT), jnp.float32),
            pltpu.SemaphoreType.DMA,
            pltpu.SemaphoreType.DMA,
            pltpu.SemaphoreType.DMA,
        ],
        compiler_params=pltpu.CompilerParams(needs_layout_passes=False),
    )(_seg_max_kernel)
    return f(node_feats, src, dst)


def kernel(node_feats, edge_index):
    ei = edge_index.astype(jnp.int32)
    return _seg_max(node_feats, ei[0], ei[1])


# speedup vs baseline: 1.0219x; 1.0219x over previous
import functools

import jax
import jax.numpy as jnp
from jax import lax
from jax.experimental import pallas as pl
from jax.experimental.pallas import tpu as pltpu
from jax.experimental.pallas import tpu_sc as plsc

N_NODES = 10000
N_EDGES = 320000
D_FEAT = 128

NC = 2
NS = 16
NW = NC * NS
RANGE = 320
TRASH = RANGE
CHUNK = 4000
NCHUNKS = N_EDGES // CHUNK
NBLOCKS_OUT = RANGE // 16
UNROLL = 5
NRING = 8

_NEG_INF = float("-inf")


def _seg_max_kernel(feats_hbm, src_hbm, dst_hbm, out_hbm,
                    dst0, src0, dst1, src1, sel, ring, acc,
                    sem, sem_e0, sem_e1):
    wid = lax.axis_index("s") * NC + lax.axis_index("c")
    lo = wid * RANGE
    hi = lo + RANGE

    neg = jnp.full((16,), _NEG_INF, jnp.float32)

    def init_body(r, _):
        for c in range(D_FEAT // 16):
            acc[r, pl.ds(c * 16, 16)] = neg
        return 0

    lax.fori_loop(0, RANGE + 1, init_body, 0)

    dummy_packed = jnp.full((16,), TRASH, jnp.int32)

    def fire_chunk(ch, dbuf, sbuf, sem_e):
        base = ch * CHUNK
        pltpu.async_copy(dst_hbm.at[pl.ds(base, CHUNK)], dbuf, sem_e)
        pltpu.async_copy(src_hbm.at[pl.ds(base, CHUNK)], sbuf, sem_e)

    def wait_chunk(dbuf, sbuf, sem_e):
        pltpu.make_async_copy(dst_hbm.at[pl.ds(0, CHUNK)], dbuf, sem_e).wait()
        pltpu.make_async_copy(src_hbm.at[pl.ds(0, CHUNK)], sbuf, sem_e).wait()

    def chunk_body(ch, dbuf, sbuf, sem_e):
        wait_chunk(dbuf, sbuf, sem_e)

        def scan_body(i, kv):
            for u in range(UNROLL):
                off = (i * UNROLL + u) * 16
                d = dbuf[pl.ds(off, 16)]
                m = (d >= lo) & (d < hi)
                s = sbuf[pl.ds(off, 16)]
                packed = s * 512 + (d - lo)
                cs = plsc.cumsum(jnp.where(m, 1, 0))
                pos = (kv - 1) + cs
                plsc.store_scatter(sel, [pos], packed, mask=m)
                kv = kv + plsc.all_reduce_population_count(m)
            return kv

        kv = lax.fori_loop(0, CHUNK // 16 // UNROLL, scan_body,
                           jnp.zeros((16,), jnp.int32))
        k = kv[0]

        pad_pos = k + lax.iota(jnp.int32, 16)
        plsc.store_scatter(sel, [pad_pos], dummy_packed)
        nblk = (k + 15) // 16

        def fire(j):
            iv = lax.shift_right_logical(sel[pl.ds(j * 16, 16)], 9)
            pltpu.async_copy(feats_hbm.at[iv], ring.at[j % NRING],
                             sem.at[j % NRING])

        def prime(j, _):
            @pl.when(j < nblk)
            def _():
                fire(j)
            return 0

        lax.fori_loop(0, NRING, prime, 0)

        def drain_body(j, _):
            slot = j % NRING
            pltpu.make_async_copy(feats_hbm.at[pl.ds(0, 16)],
                                  ring.at[slot], sem.at[slot]).wait()
            ldv = sel[pl.ds(j * 16, 16)] & 511
            lds = [ldv[r] for r in range(16)]
            for r in range(16):
                ld = lds[r]
                for c in range(D_FEAT // 16):
                    sl = pl.ds(c * 16, 16)
                    acc[ld, sl] = jnp.maximum(acc[ld, sl], ring[slot, r, sl])

            @pl.when(j + NRING < nblk)
            def _():
                fire(j + NRING)

            return 0

        lax.fori_loop(0, nblk, drain_body, 0)

    fire_chunk(0, dst0, src0, sem_e0)

    def outer_body(jj, _):
        ch0 = 2 * jj

        @pl.when(ch0 + 1 < NCHUNKS)
        def _():
            fire_chunk(ch0 + 1, dst1, src1, sem_e1)

        chunk_body(ch0, dst0, src0, sem_e0)

        @pl.when(ch0 + 2 < NCHUNKS)
        def _():
            fire_chunk(ch0 + 2, dst0, src0, sem_e0)

        @pl.when(ch0 + 1 < NCHUNKS)
        def _():
            chunk_body(ch0 + 1, dst1, src1, sem_e1)

        return 0

    lax.fori_loop(0, (NCHUNKS + 1) // 2, outer_body, 0)

    def fix_body(r, _):
        for c in range(D_FEAT // 16):
            sl = pl.ds(c * 16, 16)
            v = acc[r, sl]
            acc[r, sl] = jnp.where(v == _NEG_INF, 0.0, v)
        return 0

    lax.fori_loop(0, RANGE, fix_body, 0)

    def out_body(b, _):
        @pl.when(lo + b * 16 < N_NODES)
        def _():
            pltpu.sync_copy(acc.at[pl.ds(b * 16, 16)],
                            out_hbm.at[pl.ds(lo + b * 16, 16)])
        return 0

    lax.fori_loop(0, NBLOCKS_OUT, out_body, 0)


@jax.jit
def _seg_max(node_feats, src, dst):
    mesh = plsc.VectorSubcoreMesh(core_axis_name="c", subcore_axis_name="s")
    f = functools.partial(
        pl.kernel,
        mesh=mesh,
        out_type=jax.ShapeDtypeStruct((N_NODES, D_FEAT), jnp.float32),
        scratch_types=[
            pltpu.VMEM((CHUNK,), jnp.int32),
            pltpu.VMEM((CHUNK,), jnp.int32),
            pltpu.VMEM((CHUNK,), jnp.int32),
            pltpu.VMEM((CHUNK,), jnp.int32),
            pltpu.VMEM((CHUNK + 16,), jnp.int32),
            pltpu.VMEM((NRING, 16, D_FEAT), jnp.float32),
            pltpu.VMEM((RANGE + 1, D_FEAT), jnp.float32),
            pltpu.SemaphoreType.DMA((NRING,)),
            pltpu.SemaphoreType.DMA,
            pltpu.SemaphoreType.DMA,
        ],
        compiler_params=pltpu.CompilerParams(needs_layout_passes=False),
    )(_seg_max_kernel)
    return f(node_feats, src, dst)


def kernel(node_feats, edge_index):
    ei = edge_index.astype(jnp.int32)
    return _seg_max(node_feats, ei[0], ei[1])
